# baseline (speedup 1.0000x reference)
import functools

import jax
import jax.numpy as jnp
from jax import lax
from jax.experimental import pallas as pl
from jax.experimental.pallas import tpu as pltpu
from jax.experimental.pallas import tpu_sc as plsc

B = 16384
D = 128
NC = 2
NS = 16
NW = NC * NS
CH = 128
CHUNKS = 2
NB = B // CHUNKS

_mesh = plsc.VectorSubcoreMesh(core_axis_name="c", subcore_axis_name="s")


def _make_sc_gather(nb):
    bpw = nb // NW
    nch = bpw // CH

    @functools.partial(
        pl.kernel,
        out_type=(
            jax.ShapeDtypeStruct((nb, D), jnp.float32),
            jax.ShapeDtypeStruct((nb, D), jnp.float32),
        ),
        mesh=_mesh,
        scratch_types=[
            pltpu.VMEM((nch, CH), jnp.int32),
            pltpu.VMEM((bpw, D), jnp.float32),
            pltpu.SemaphoreType.DMA,
        ],
    )
    def _sc_gather(uid_hbm, iid_hbm, ut_hbm, it_hbm, uout_hbm, iout_hbm,
                   idx_v, rows_v, sem):
        wid = lax.axis_index("s") * NC + lax.axis_index("c")
        base = wid * bpw
        for ids_hbm, table_hbm, out_hbm in (
            (uid_hbm, ut_hbm, uout_hbm),
            (iid_hbm, it_hbm, iout_hbm),
        ):
            pltpu.sync_copy(ids_hbm.at[wid], idx_v)
            copies = [
                pltpu.async_copy(
                    table_hbm.at[idx_v.at[j]],
                    rows_v.at[pl.ds(j * CH, CH)],
                    sem,
                )
                for j in range(nch)
            ]
            for c in copies:
                c.wait()
            pltpu.sync_copy(rows_v, out_hbm.at[pl.ds(base, bpw)])

    return _sc_gather


def _mlp_body(u_ref, i_ref, w1a_ref, w1b_ref, b1_ref, w2_ref, b2_ref,
              w3_ref, b3_ref, o_ref):
    h1 = jnp.dot(u_ref[...], w1a_ref[...], preferred_element_type=jnp.float32)
    h1 += jnp.dot(i_ref[...], w1b_ref[...], preferred_element_type=jnp.float32)
    h1 = jnp.maximum(h1 + b1_ref[...], 0.0)
    h2 = jnp.maximum(
        jnp.dot(h1, w2_ref[...], preferred_element_type=jnp.float32)
        + b2_ref[...], 0.0)
    o = jnp.maximum(
        jnp.dot(h2, w3_ref[...], preferred_element_type=jnp.float32)
        + b3_ref[...], 0.0)
    o_ref[...] = o


def _make_mlp(nb, bm):
    return pl.pallas_call(
        _mlp_body,
        grid=(nb // bm,),
        in_specs=[
            pl.BlockSpec((bm, D), lambda i: (i, 0)),
            pl.BlockSpec((bm, D), lambda i: (i, 0)),
            pl.BlockSpec((D, 64), lambda i: (0, 0)),
            pl.BlockSpec((D, 64), lambda i: (0, 0)),
            pl.BlockSpec((1, 64), lambda i: (0, 0)),
            pl.BlockSpec((64, 32), lambda i: (0, 0)),
            pl.BlockSpec((1, 32), lambda i: (0, 0)),
            pl.BlockSpec((32, 1), lambda i: (0, 0)),
            pl.BlockSpec((1, 1), lambda i: (0, 0)),
        ],
        out_specs=pl.BlockSpec((bm, 1), lambda i: (i, 0)),
        out_shape=jax.ShapeDtypeStruct((nb, 1), jnp.float32),
    )


_sc_gather = _make_sc_gather(NB)
_mlp = _make_mlp(NB, 2048)


def kernel(user_id, item_id, user_table, item_table, W1, b1, W2, b2, W3, b3):
    bpw = NB // NW
    nch = bpw // CH
    uid = user_id.astype(jnp.int32).reshape(CHUNKS, NW, nch, CH)
    iid = item_id.astype(jnp.int32).reshape(CHUNKS, NW, nch, CH)
    w1a, w1b = W1[:D], W1[D:]
    b1r, b2r, b3r = b1.reshape(1, 64), b2.reshape(1, 32), b3.reshape(1, 1)
    embs = [_sc_gather(uid[c], iid[c], user_table, item_table)
            for c in range(CHUNKS)]
    outs = [_mlp(u_emb, i_emb, w1a, w1b, b1r, W2, b2r, W3, b3r)
            for (u_emb, i_emb) in embs]
    return jnp.concatenate(outs, axis=0).reshape(-1)
